# baseline (speedup 1.0000x reference)
import functools

import jax
import jax.numpy as jnp
from jax import lax
from jax.experimental import pallas as pl
from jax.experimental.pallas import tpu as pltpu
from jax.experimental.pallas import tpu_sc as plsc

NC = 2
NS = 16
NW = NC * NS
LANES = 16
CHUNK = 125
N_PAD = 10240
RING = 2
UNIT = 40
UNITS_PER_TILE = 2


def _sc_agg(x, edges3, n_units):
    cin = x.shape[1]
    rows_per_sub = N_PAD // NS
    assert n_units == UNITS_PER_TILE * NW
    n_groups = UNIT // RING

    mesh = plsc.VectorSubcoreMesh(core_axis_name="c", subcore_axis_name="s")

    @functools.partial(
        pl.kernel,
        out_type=jax.ShapeDtypeStruct((NC, N_PAD, cin), jnp.float32),
        mesh=mesh,
        scratch_types=[
            pltpu.VMEM((UNIT, CHUNK), jnp.int32),
            pltpu.VMEM((UNIT, CHUNK), jnp.int32),
            pltpu.VMEM((UNIT, cin), jnp.float32),
            *[pltpu.VMEM((CHUNK, cin), jnp.float32) for _ in range(RING)],
            pltpu.VMEM_SHARED((N_PAD, cin), jnp.float32),
            *[pltpu.SemaphoreType.DMA for _ in range(2 * RING)],
        ],
    )
    def body(x_hbm, edges_hbm, out_hbm, src_v, dst_v, zbuf, *rest):
        rows_v = rest[:RING]
        agg_sh = rest[RING]
        gsem = rest[RING + 1:RING + 1 + RING]
        ssem = rest[RING + 1 + RING:RING + 1 + 2 * RING]
        c = lax.axis_index("c")
        s = lax.axis_index("s")
        wid = c * NS + s

        def gather(j, b):
            pltpu.async_copy(x_hbm.at[src_v.at[j]], rows_v[b], gsem[b])

        def gather_wait(b):
            pltpu.make_async_copy(x_hbm.at[src_v.at[0]], rows_v[b],
                                  gsem[b]).wait()

        def scatter(j, b):
            pltpu.async_copy(rows_v[b], agg_sh.at[dst_v.at[j]], ssem[b],
                             add=True)

        def scatter_wait(b):
            pltpu.make_async_copy(rows_v[b], agg_sh.at[dst_v.at[0]],
                                  ssem[b]).wait()

        def stage_unit(unit):
            c0 = unit * UNIT
            pltpu.sync_copy(edges_hbm.at[0, pl.ds(c0, UNIT)], src_v)
            pltpu.sync_copy(edges_hbm.at[1, pl.ds(c0, UNIT)], dst_v)

        stage_unit(wid)
        gather(0, 0)
        gather(1, 1)

        def zrow(i, _):
            def zcol(k, __):
                zbuf[i, pl.ds(k * LANES, LANES)] = jnp.zeros(
                    (LANES,), jnp.float32)
                return 0
            return lax.fori_loop(0, cin // LANES, zcol, 0)
        lax.fori_loop(0, UNIT, zrow, 0)

        base = s * rows_per_sub
        for m in range(rows_per_sub // UNIT):
            pltpu.sync_copy(zbuf, agg_sh.at[pl.ds(base + m * UNIT, UNIT)])
        plsc.subcore_barrier()

        def pipeline():
            def group(g, _):
                j = g * RING
                gather_wait(0)
                scatter(j, 0)
                gather_wait(1)
                scatter(j + 1, 1)
                scatter_wait(0)
                gather(j + RING, 0)
                scatter_wait(1)
                gather(j + RING + 1, 1)
                return 0
            lax.fori_loop(0, n_groups - 1, group, 0)
            jlast = (n_groups - 1) * RING
            gather_wait(0)
            scatter(jlast, 0)
            gather_wait(1)
            scatter(jlast + 1, 1)
            scatter_wait(0)
            scatter_wait(1)

        pipeline()
        for p in range(1, UNITS_PER_TILE):
            stage_unit(p * NW + wid)
            gather(0, 0)
            gather(1, 1)
            pipeline()

        plsc.subcore_barrier()
        pltpu.sync_copy(agg_sh.at[pl.ds(base, rows_per_sub)],
                        out_hbm.at[c, pl.ds(base, rows_per_sub)])

    return body(x, edges3)


def _tc_combine(parts, x, w_rel, w_root):
    n, cin = x.shape
    cout = w_rel.shape[0]
    bm = 1000

    def body(a0_ref, a1_ref, x_ref, wr_ref, wo_ref, o_ref):
        dnums = (((1,), (1,)), ((), ()))
        agg = a0_ref[0] + a1_ref[0]
        acc = lax.dot_general(agg, wr_ref[...], dnums,
                              preferred_element_type=jnp.float32)
        acc = acc + lax.dot_general(x_ref[...], wo_ref[...], dnums,
                                    preferred_element_type=jnp.float32)
        o_ref[...] = jnp.maximum(acc, 0.0)

    return pl.pallas_call(
        body,
        grid=(n // bm,),
        in_specs=[
            pl.BlockSpec((1, bm, cin), lambda i: (0, i, 0)),
            pl.BlockSpec((1, bm, cin), lambda i: (1, i, 0)),
            pl.BlockSpec((bm, cin), lambda i: (i, 0)),
            pl.BlockSpec((cout, cin), lambda i: (0, 0)),
            pl.BlockSpec((cout, cin), lambda i: (0, 0)),
        ],
        out_specs=pl.BlockSpec((bm, cout), lambda i: (i, 0)),
        out_shape=jax.ShapeDtypeStruct((n, cout), jnp.float32),
    )(parts, parts, x, w_rel, w_root)


def kernel(x, edge_index, W_rel, W_root):
    n = x.shape[0]
    e = edge_index.shape[1]

    n_units = UNITS_PER_TILE * NW
    assert n_units * UNIT * CHUNK == e
    edges3 = edge_index.astype(jnp.int32).reshape(2, e // CHUNK, CHUNK)

    parts = _sc_agg(x, edges3, n_units)
    return _tc_combine(parts, x, W_rel, W_root)


# speedup vs baseline: 1.2517x; 1.2517x over previous
import functools

import jax
import jax.numpy as jnp
from jax import lax
from jax.experimental import pallas as pl
from jax.experimental.pallas import tpu as pltpu
from jax.experimental.pallas import tpu_sc as plsc

NC = 2
NS = 16
NW = NC * NS
LANES = 16
CHUNK = 125
N_PAD = 10240
RING = 2
UNIT = 40
UNITS_PER_TILE = 2


def _sc_agg(x, edges3, n_units):
    cin = x.shape[1]
    rows_per_sub = N_PAD // NS
    assert n_units == UNITS_PER_TILE * NW
    n_groups = UNIT // RING

    mesh = plsc.VectorSubcoreMesh(core_axis_name="c", subcore_axis_name="s")

    @functools.partial(
        pl.kernel,
        out_type=jax.ShapeDtypeStruct((NC, N_PAD, cin), jnp.float32),
        mesh=mesh,
        scratch_types=[
            pltpu.VMEM((UNIT, CHUNK), jnp.int32),
            pltpu.VMEM((UNIT, CHUNK), jnp.int32),
            pltpu.VMEM((UNIT, cin), jnp.float32),
            *[pltpu.VMEM((CHUNK, cin), jnp.float32) for _ in range(RING)],
            pltpu.VMEM_SHARED((N_PAD, cin), jnp.float32),
            *[pltpu.SemaphoreType.DMA for _ in range(RING)],
        ],
    )
    def body(x_hbm, edges_hbm, out_hbm, src_v, dst_v, zbuf, *rest):
        rows_v = rest[:RING]
        agg_sh = rest[RING]
        gsem = rest[RING + 1:RING + 1 + RING]
        c = lax.axis_index("c")
        s = lax.axis_index("s")
        wid = c * NS + s

        def gather(j, b):
            pltpu.async_copy(x_hbm.at[src_v.at[j]], rows_v[b], gsem[b])

        def gather_wait(b):
            pltpu.make_async_copy(x_hbm.at[src_v.at[0]], rows_v[b],
                                  gsem[b]).wait()

        def scatter(j, b):
            pltpu.sync_copy(rows_v[b], agg_sh.at[dst_v.at[j]], add=True)

        def stage_unit(unit):
            c0 = unit * UNIT
            pltpu.sync_copy(edges_hbm.at[0, pl.ds(c0, UNIT)], src_v)
            pltpu.sync_copy(edges_hbm.at[1, pl.ds(c0, UNIT)], dst_v)

        stage_unit(wid)
        gather(0, 0)
        gather(1, 1)

        def zrow(i, _):
            def zcol(k, __):
                zbuf[i, pl.ds(k * LANES, LANES)] = jnp.zeros(
                    (LANES,), jnp.float32)
                return 0
            return lax.fori_loop(0, cin // LANES, zcol, 0)
        lax.fori_loop(0, UNIT, zrow, 0)

        base = s * rows_per_sub
        for m in range(rows_per_sub // UNIT):
            pltpu.sync_copy(zbuf, agg_sh.at[pl.ds(base + m * UNIT, UNIT)])
        plsc.subcore_barrier()

        def pipeline():
            def group(g, _):
                j = g * RING
                gather_wait(0)
                scatter(j, 0)
                gather(j + RING, 0)
                gather_wait(1)
                scatter(j + 1, 1)
                gather(j + RING + 1, 1)
                return 0
            lax.fori_loop(0, n_groups - 1, group, 0)
            jlast = (n_groups - 1) * RING
            gather_wait(0)
            scatter(jlast, 0)
            gather_wait(1)
            scatter(jlast + 1, 1)

        pipeline()
        for p in range(1, UNITS_PER_TILE):
            stage_unit(p * NW + wid)
            gather(0, 0)
            gather(1, 1)
            pipeline()

        plsc.subcore_barrier()
        pltpu.sync_copy(agg_sh.at[pl.ds(base, rows_per_sub)],
                        out_hbm.at[c, pl.ds(base, rows_per_sub)])

    return body(x, edges3)


def _tc_combine(parts, x, w_rel, w_root):
    n, cin = x.shape
    cout = w_rel.shape[0]
    bm = 1000

    def body(a0_ref, a1_ref, x_ref, wr_ref, wo_ref, o_ref):
        dnums = (((1,), (1,)), ((), ()))
        agg = a0_ref[0] + a1_ref[0]
        acc = lax.dot_general(agg, wr_ref[...], dnums,
                              preferred_element_type=jnp.float32)
        acc = acc + lax.dot_general(x_ref[...], wo_ref[...], dnums,
                                    preferred_element_type=jnp.float32)
        o_ref[...] = jnp.maximum(acc, 0.0)

    return pl.pallas_call(
        body,
        grid=(n // bm,),
        in_specs=[
            pl.BlockSpec((1, bm, cin), lambda i: (0, i, 0)),
            pl.BlockSpec((1, bm, cin), lambda i: (1, i, 0)),
            pl.BlockSpec((bm, cin), lambda i: (i, 0)),
            pl.BlockSpec((cout, cin), lambda i: (0, 0)),
            pl.BlockSpec((cout, cin), lambda i: (0, 0)),
        ],
        out_specs=pl.BlockSpec((bm, cout), lambda i: (i, 0)),
        out_shape=jax.ShapeDtypeStruct((n, cout), jnp.float32),
    )(parts, parts, x, w_rel, w_root)


def kernel(x, edge_index, W_rel, W_root):
    n = x.shape[0]
    e = edge_index.shape[1]

    n_units = UNITS_PER_TILE * NW
    assert n_units * UNIT * CHUNK == e
    edges3 = edge_index.astype(jnp.int32).reshape(2, e // CHUNK, CHUNK)

    parts = _sc_agg(x, edges3, n_units)
    return _tc_combine(parts, x, W_rel, W_root)
